# initial kernel scaffold (unmeasured)
import jax
import jax.numpy as jnp
from jax import lax
from jax.experimental import pallas as pl
from jax.experimental.pallas import tpu as pltpu

N_DEV = 16
N_TOK = 256
ROWS_PER_DEV = N_TOK // N_DEV
N_EXP = 32
EXP_PER_DEV = N_EXP // N_DEV
D_OUT = 256


def kernel(x, router_W, route_idx, expert_W, shared_W):
    def body(x_ref, rw_ref, idx_ref, ew_ref, sw_ref, out_ref,
             send_buf, recv_buf, send_sems, recv_sems):
        my = lax.axis_index("i")

        scores = jnp.dot(x_ref[:, :], rw_ref[:, :],
                         preferred_element_type=jnp.float32)
        s_max = jnp.max(scores, axis=-1, keepdims=True)
        ex = jnp.exp(scores - s_max)
        probs = ex / jnp.sum(ex, axis=-1, keepdims=True)
        idx = idx_ref[:, :]
        col = lax.broadcasted_iota(jnp.int32, (N_TOK, N_EXP), 1)
        p_sel = jnp.sum(jnp.where(col == idx, probs, 0.0),
                        axis=-1, keepdims=True)

        e0 = my * EXP_PER_DEV
        m0 = jnp.where(idx == e0, p_sel, 0.0)
        m1 = jnp.where(idx == e0 + 1, p_sel, 0.0)
        xv = x_ref[:, :]
        send_buf[:, :] = (
            jnp.dot(m0 * xv, ew_ref[0, :, :],
                    preferred_element_type=jnp.float32)
            + jnp.dot(m1 * xv, ew_ref[1, :, :],
                      preferred_element_type=jnp.float32)
        )

        recv_buf[0, :, :] = send_buf[pl.ds(my * ROWS_PER_DEV, ROWS_PER_DEV), :]

        rdmas = []
        for o in range(1, N_DEV):
            dest = lax.rem(my + o, N_DEV)
            rdma = pltpu.make_async_remote_copy(
                src_ref=send_buf.at[pl.ds(dest * ROWS_PER_DEV, ROWS_PER_DEV), :],
                dst_ref=recv_buf.at[o],
                send_sem=send_sems.at[o],
                recv_sem=recv_sems.at[o],
                device_id=(dest,),
                device_id_type=pl.DeviceIdType.MESH,
            )
            rdma.start()
            rdmas.append(rdma)

        x_my = x_ref[pl.ds(my * ROWS_PER_DEV, ROWS_PER_DEV), :]
        shared = jnp.dot(x_my, sw_ref[:, :], preferred_element_type=jnp.float32)

        for r in rdmas:
            r.wait_recv()

        out_ref[:, :] = shared + jnp.sum(recv_buf[:, :, :], axis=0)

        for r in rdmas:
            r.wait_send()

    return pl.pallas_call(
        body,
        out_shape=jax.ShapeDtypeStruct((ROWS_PER_DEV, D_OUT), jnp.float32),
        in_specs=[pl.BlockSpec(memory_space=pltpu.VMEM)] * 5,
        out_specs=pl.BlockSpec(memory_space=pltpu.VMEM),
        scratch_shapes=[
            pltpu.VMEM((N_TOK, D_OUT), jnp.float32),
            pltpu.VMEM((N_DEV, ROWS_PER_DEV, D_OUT), jnp.float32),
            pltpu.SemaphoreType.DMA((N_DEV,)),
            pltpu.SemaphoreType.DMA((N_DEV,)),
        ],
        compiler_params=pltpu.CompilerParams(collective_id=0),
    )(x, router_W, route_idx, expert_W, shared_W)


# baseline (device time: 18790 ns/iter reference)
import jax
import jax.numpy as jnp
from jax import lax
from jax.experimental import pallas as pl
from jax.experimental.pallas import tpu as pltpu

N_DEV = 16
N_TOK = 256
ROWS_PER_DEV = N_TOK // N_DEV
N_EXP = 32
EXP_PER_DEV = N_EXP // N_DEV
D_OUT = 256


def kernel(x, router_W, route_idx, expert_W, shared_W):
    def body(x_ref, rw_ref, idx_ref, ew_ref, sw_ref, out_ref,
             send_buf, recv_buf, send_sems, recv_sems):
        my = lax.axis_index("i")

        scores = jnp.dot(x_ref[:, :], rw_ref[:, :],
                         preferred_element_type=jnp.float32)
        s_max = jnp.max(scores, axis=-1, keepdims=True)
        ex = jnp.exp(scores - s_max)
        probs = ex / jnp.sum(ex, axis=-1, keepdims=True)
        idx = idx_ref[:, :]
        col = lax.broadcasted_iota(jnp.int32, (N_TOK, N_EXP), 1)
        p_sel = jnp.sum(jnp.where(col == idx, probs, 0.0),
                        axis=-1, keepdims=True)

        e0 = my * EXP_PER_DEV
        m0 = jnp.where(idx == e0, p_sel, 0.0)
        m1 = jnp.where(idx == e0 + 1, p_sel, 0.0)
        xv = x_ref[:, :]
        send_buf[:, :] = (
            jnp.dot(m0 * xv, ew_ref[0, :, :],
                    preferred_element_type=jnp.float32)
            + jnp.dot(m1 * xv, ew_ref[1, :, :],
                      preferred_element_type=jnp.float32)
        )

        recv_buf[0, :, :] = send_buf[pl.ds(my * ROWS_PER_DEV, ROWS_PER_DEV), :]

        rdmas = []
        for o in range(1, N_DEV):
            dest = lax.rem(my + o, N_DEV)
            rdma = pltpu.make_async_remote_copy(
                src_ref=send_buf.at[pl.ds(dest * ROWS_PER_DEV, ROWS_PER_DEV), :],
                dst_ref=recv_buf.at[o],
                send_sem=send_sems.at[o],
                recv_sem=recv_sems.at[o],
                device_id=(dest,),
                device_id_type=pl.DeviceIdType.MESH,
            )
            rdma.start()
            rdmas.append(rdma)

        x_my = x_ref[pl.ds(my * ROWS_PER_DEV, ROWS_PER_DEV), :]
        shared = jnp.dot(x_my, sw_ref[:, :], preferred_element_type=jnp.float32)

        for r in rdmas:
            r.wait_recv()

        out_ref[:, :] = shared + jnp.sum(recv_buf[:, :, :], axis=0)

        for r in rdmas:
            r.wait_send()

    return pl.pallas_call(
        body,
        out_shape=jax.ShapeDtypeStruct((ROWS_PER_DEV, D_OUT), jnp.float32),
        in_specs=[pl.BlockSpec(memory_space=pltpu.VMEM)] * 5,
        out_specs=pl.BlockSpec(memory_space=pltpu.VMEM),
        scratch_shapes=[
            pltpu.VMEM((N_TOK, D_OUT), jnp.float32),
            pltpu.VMEM((N_DEV, ROWS_PER_DEV, D_OUT), jnp.float32),
            pltpu.SemaphoreType.DMA((N_DEV,)),
            pltpu.SemaphoreType.DMA((N_DEV,)),
        ],
    )(x, router_W, route_idx, expert_W, shared_W)


# device time: 14211 ns/iter; 1.3222x vs baseline; 1.3222x over previous
import jax
import jax.numpy as jnp
from jax import lax
from jax.experimental import pallas as pl
from jax.experimental.pallas import tpu as pltpu

N_DEV = 16
N_TOK = 256
ROWS_PER_DEV = N_TOK // N_DEV
N_EXP = 32
EXP_PER_DEV = N_EXP // N_DEV
D_OUT = 256


def kernel(x, router_W, route_idx, expert_W, shared_W):
    def body(x_ref, rw_ref, idx_ref, ew_ref, sw_ref, out_ref,
             send_buf, recv_buf, send_sems, recv_sems):
        my = lax.axis_index("i")

        scores = jnp.dot(x_ref[:, :], rw_ref[:, :],
                         preferred_element_type=jnp.float32)
        s_max = jnp.max(scores, axis=-1, keepdims=True)
        ex = jnp.exp(scores - s_max)
        probs = ex / jnp.sum(ex, axis=-1, keepdims=True)
        idx = idx_ref[:, :]
        col = lax.broadcasted_iota(jnp.int32, (N_TOK, N_EXP), 1)
        p_sel = jnp.sum(jnp.where(col == idx, probs, 0.0),
                        axis=-1, keepdims=True)

        e0 = my * EXP_PER_DEV
        m0 = jnp.where(idx == e0, p_sel, 0.0)
        m1 = jnp.where(idx == e0 + 1, p_sel, 0.0)
        xv = x_ref[:, :]
        send_buf[:, :] = (
            jnp.dot(m0 * xv, ew_ref[0, :, :],
                    preferred_element_type=jnp.float32)
            + jnp.dot(m1 * xv, ew_ref[1, :, :],
                      preferred_element_type=jnp.float32)
        )

        barrier_sem = pltpu.get_barrier_semaphore()
        for o in range(1, N_DEV):
            peer = lax.rem(my + o, N_DEV)
            pl.semaphore_signal(barrier_sem, inc=1, device_id=(peer,),
                                device_id_type=pl.DeviceIdType.MESH)
        pl.semaphore_wait(barrier_sem, N_DEV - 1)

        rdmas = []
        for o in range(1, N_DEV):
            dest = lax.rem(my + o, N_DEV)
            rdma = pltpu.make_async_remote_copy(
                src_ref=send_buf.at[pl.ds(dest * ROWS_PER_DEV, ROWS_PER_DEV), :],
                dst_ref=recv_buf.at[o],
                send_sem=send_sems.at[o],
                recv_sem=recv_sems.at[o],
                device_id=(dest,),
                device_id_type=pl.DeviceIdType.MESH,
            )
            rdma.start()
            rdmas.append(rdma)

        x_my = x_ref[pl.ds(my * ROWS_PER_DEV, ROWS_PER_DEV), :]
        shared = jnp.dot(x_my, sw_ref[:, :], preferred_element_type=jnp.float32)

        for r in rdmas:
            r.wait_recv()

        own = send_buf[pl.ds(my * ROWS_PER_DEV, ROWS_PER_DEV), :]
        out_ref[:, :] = shared + own + jnp.sum(recv_buf[1:, :, :], axis=0)

        for r in rdmas:
            r.wait_send()

    return pl.pallas_call(
        body,
        out_shape=jax.ShapeDtypeStruct((ROWS_PER_DEV, D_OUT), jnp.float32),
        in_specs=[pl.BlockSpec(memory_space=pltpu.VMEM)] * 5,
        out_specs=pl.BlockSpec(memory_space=pltpu.VMEM),
        scratch_shapes=[
            pltpu.VMEM((N_TOK, D_OUT), jnp.float32),
            pltpu.VMEM((N_DEV, ROWS_PER_DEV, D_OUT), jnp.float32),
            pltpu.SemaphoreType.DMA((N_DEV,)),
            pltpu.SemaphoreType.DMA((N_DEV,)),
        ],
        compiler_params=pltpu.CompilerParams(collective_id=0),
    )(x, router_W, route_idx, expert_W, shared_W)
